# baseline (speedup 1.0000x reference)
import jax
from jax.experimental import pallas as pl

_BLKC = 2560


def _slice_transpose_kernel(xt_ref, o_ref):
    o_ref[...] = xt_ref[3:, :].T


def kernel(x, W, b):
    n, f = x.shape
    fo = f - 3
    xt = x.T
    return pl.pallas_call(
        _slice_transpose_kernel,
        grid=(pl.cdiv(n, _BLKC),),
        in_specs=[pl.BlockSpec((f, _BLKC), lambda i: (0, i))],
        out_specs=pl.BlockSpec((_BLKC, fo), lambda i: (i, 0)),
        out_shape=jax.ShapeDtypeStruct((n, fo), x.dtype),
    )(xt)


# speedup vs baseline: 1.1927x; 1.1927x over previous
import jax
from jax.experimental import pallas as pl
from jax.experimental.pallas import tpu as pltpu


def _kern(xt_hbm, o_hbm, vin, vout, sin, sout):
    f, n = vin.shape
    cin = pltpu.make_async_copy(xt_hbm, vin, sin)
    cin.start()
    cin.wait()
    chunks = []
    c0 = 0
    while c0 < n:
        sz = min(1024, n - c0)
        chunks.append((c0, sz))
        c0 += sz
    for i, (c0, sz) in enumerate(chunks):
        vout[pl.ds(c0, sz), :] = vin[3:, c0:c0 + sz].T
        pltpu.make_async_copy(
            vout.at[pl.ds(c0, sz), :], o_hbm.at[pl.ds(c0, sz), :], sout.at[i]
        ).start()
    for i, (c0, sz) in enumerate(chunks):
        pltpu.make_async_copy(
            vout.at[pl.ds(c0, sz), :], o_hbm.at[pl.ds(c0, sz), :], sout.at[i]
        ).wait()


def kernel(x, W, b):
    n, f = x.shape
    fo = f - 3
    xt = x.T
    nchunks = -(-n // 1024)
    return pl.pallas_call(
        _kern,
        in_specs=[pl.BlockSpec(memory_space=pltpu.MemorySpace.HBM)],
        out_specs=pl.BlockSpec(memory_space=pltpu.MemorySpace.HBM),
        out_shape=jax.ShapeDtypeStruct((n, fo), x.dtype),
        scratch_shapes=[
            pltpu.VMEM((f, n), x.dtype),
            pltpu.VMEM((n, fo), x.dtype),
            pltpu.SemaphoreType.DMA,
            pltpu.SemaphoreType.DMA((nchunks,)),
        ],
    )(xt)


# speedup vs baseline: 1.1945x; 1.0015x over previous
import jax
from jax.experimental import pallas as pl
from jax.experimental.pallas import tpu as pltpu


def _kern(xt_ref, o_hbm, vout, sout):
    f, n = xt_ref.shape
    chunks = []
    c0 = 0
    while c0 < n:
        sz = min(1024, n - c0)
        chunks.append((c0, sz))
        c0 += sz
    for i, (c0, sz) in enumerate(chunks):
        vout[pl.ds(c0, sz), :] = xt_ref[3:, c0:c0 + sz].T
        pltpu.make_async_copy(
            vout.at[pl.ds(c0, sz), :], o_hbm.at[pl.ds(c0, sz), :], sout.at[i]
        ).start()
    for i, (c0, sz) in enumerate(chunks):
        pltpu.make_async_copy(
            vout.at[pl.ds(c0, sz), :], o_hbm.at[pl.ds(c0, sz), :], sout.at[i]
        ).wait()


def kernel(x, W, b):
    n, f = x.shape
    fo = f - 3
    xt = x.T
    nchunks = -(-n // 1024)
    return pl.pallas_call(
        _kern,
        in_specs=[pl.BlockSpec(memory_space=pltpu.MemorySpace.VMEM)],
        out_specs=pl.BlockSpec(memory_space=pltpu.MemorySpace.HBM),
        out_shape=jax.ShapeDtypeStruct((n, fo), x.dtype),
        scratch_shapes=[
            pltpu.VMEM((n, fo), x.dtype),
            pltpu.SemaphoreType.DMA((nchunks,)),
        ],
    )(xt)


# speedup vs baseline: 1.2359x; 1.0346x over previous
import jax
from jax.experimental import pallas as pl
from jax.experimental.pallas import tpu as pltpu

_BLKC = 5120


def _slice_transpose_kernel(xt_ref, o_ref):
    o_ref[...] = xt_ref[3:, :].T


def kernel(x, W, b):
    n, f = x.shape
    fo = f - 3
    xt = x.T
    return pl.pallas_call(
        _slice_transpose_kernel,
        grid=(pl.cdiv(n, _BLKC),),
        in_specs=[pl.BlockSpec((f, _BLKC), lambda i: (0, i))],
        out_specs=pl.BlockSpec((_BLKC, fo), lambda i: (i, 0)),
        out_shape=jax.ShapeDtypeStruct((n, fo), x.dtype),
        compiler_params=pltpu.CompilerParams(dimension_semantics=("parallel",)),
    )(xt)
